# baseline (speedup 1.0000x reference)
import jax
import jax.numpy as jnp
import numpy as np
from jax.experimental import pallas as pl
from jax.experimental.pallas import tpu as pltpu

_VMEM_LIMIT = 100 * 1024 * 1024

_U1 = [(np.arange(28)[None, :, None]
        == 2 * np.arange(12)[None, None, :] + rh
        + np.arange(5)[:, None, None]).astype(np.float32) for rh in (0, 1)]

_OH1 = [(np.arange(28)[None, :, None]
         == 2 * (np.arange(120)[None, None, :] % 12) + rw
         + np.arange(5)[:, None, None]).astype(np.float32) for rw in (0, 1)]

_U2 = (np.arange(12)[None, :, None]
       == np.arange(10)[None, None, :]
       + np.arange(3)[:, None, None]).astype(np.float32)

_OH2 = (np.arange(120)[None, :, None] % 12
        == np.arange(200)[None, None, :] % 10
        + np.arange(3)[:, None, None]).astype(np.float32)


def _build_a1f(w1):
    w1k = w1.reshape(10, 5, 5).transpose(1, 2, 0)
    e1 = jnp.repeat(w1k, 12, axis=2)
    mats = []
    for rh in (0, 1):
        u = _U1[rh].reshape(5, 336)
        for rw in (0, 1):
            a1 = jnp.einsum("hwc,wic->hic", e1, _OH1[rw])
            m = jnp.dot(u.T, a1.reshape(5, 3360))
            m = m.astype(jnp.bfloat16)
            m = m.reshape(28, 12, 28, 120).transpose(0, 2, 1, 3)
            mats.append(m.reshape(784, 1440))
    return jnp.stack(mats, axis=0)


def _build_a2f(w2):
    w2k = w2.reshape(20, 10, 3, 3).transpose(2, 3, 1, 0)
    e2 = jnp.repeat(jnp.repeat(w2k, 12, axis=2), 10, axis=3)
    a2 = jnp.einsum("hwrc,wrc->hrc", e2, _OH2)
    m = jnp.dot(_U2.reshape(3, 120).T, a2.reshape(3, 24000))
    m = m.astype(jnp.bfloat16)
    m = m.reshape(12, 10, 24000).transpose(1, 0, 2)
    m = m.reshape(10, 1440, 200).transpose(1, 0, 2)
    return m.reshape(1440, 2000)


def _lenet_kernel(x_ref, a1f_ref, b1c_ref, a2f_ref, b2c_ref, w1p_ref, bf1_ref,
                  wf2_ref, bf2_ref, o_ref):
    x = x_ref[...].astype(jnp.bfloat16)
    t = None
    for rs in range(4):
        d = jnp.dot(x, a1f_ref[rs], preferred_element_type=jnp.float32)
        t = d if t is None else jnp.maximum(t, d)
    t = jnp.maximum(t + b1c_ref[...], 0.0)
    t = t.astype(jnp.bfloat16)
    u = jnp.dot(t, a2f_ref[...], preferred_element_type=jnp.float32)
    u = jnp.maximum(u + b2c_ref[...], 0.0)
    h = jnp.dot(u.astype(jnp.bfloat16), w1p_ref[...],
                preferred_element_type=jnp.float32)
    h = jnp.maximum(h + bf1_ref[...], 0.0)
    logits = jnp.dot(h.astype(jnp.bfloat16), wf2_ref[...],
                     preferred_element_type=jnp.float32)
    logits = logits + bf2_ref[...]
    m = jnp.max(logits, axis=-1, keepdims=True)
    s = logits - m
    lse = jnp.log(jnp.sum(jnp.exp(s), axis=-1, keepdims=True))
    o_ref[...] = (s - lse).astype(o_ref.dtype)


def kernel(w1, b1, w2, b2, wf1t, bf1, wf2t, bf2, x):
    batch = x.shape[0]
    tb = 256 if batch % 256 == 0 else (128 if batch % 128 == 0 else batch)
    xf = x.reshape(batch, 28 * 28)
    a1f = _build_a1f(w1)
    a2f = _build_a2f(w2)
    b1c = jnp.tile(jnp.repeat(b1.reshape(10), 12), 12).reshape(1, 1440)
    b2c = jnp.tile(jnp.repeat(b2.reshape(20), 10), 10).reshape(1, 2000)
    w1p = (wf1t.reshape(20, 10, 10, 500).transpose(1, 0, 2, 3)
           .reshape(2000, 500).astype(jnp.bfloat16))
    wf2b = wf2t.astype(jnp.bfloat16)
    cost = pl.CostEstimate(
        flops=2 * batch * (4 * 784 * 1440 + 10 * (1440 * 200 + 200 * 500)
                           + 500 * 10),
        transcendentals=batch * 11,
        bytes_accessed=4 * (xf.size + batch * 10 + a1f.size + a2f.size
                            + w1p.size + wf2t.size),
    )
    return pl.pallas_call(
        _lenet_kernel,
        out_shape=jax.ShapeDtypeStruct((batch, 10), x.dtype),
        grid=(batch // tb,),
        in_specs=[
            pl.BlockSpec((tb, 784), lambda i: (i, 0)),
            pl.BlockSpec((4, 784, 1440), lambda i: (0, 0, 0)),
            pl.BlockSpec((1, 1440), lambda i: (0, 0)),
            pl.BlockSpec((1440, 2000), lambda i: (0, 0)),
            pl.BlockSpec((1, 2000), lambda i: (0, 0)),
            pl.BlockSpec((2000, 500), lambda i: (0, 0)),
            pl.BlockSpec((1, 500), lambda i: (0, 0)),
            pl.BlockSpec((500, 10), lambda i: (0, 0)),
            pl.BlockSpec((1, 10), lambda i: (0, 0)),
        ],
        out_specs=pl.BlockSpec((tb, 10), lambda i: (i, 0)),
        compiler_params=pltpu.CompilerParams(
            dimension_semantics=("parallel",),
            vmem_limit_bytes=_VMEM_LIMIT,
        ),
        cost_estimate=cost,
    )(xf, a1f, b1c, a2f, b2c, w1p, bf1, wf2b, bf2)


# speedup vs baseline: 1.1579x; 1.1579x over previous
import jax
import jax.numpy as jnp
import numpy as np
from jax.experimental import pallas as pl
from jax.experimental.pallas import tpu as pltpu

_VMEM_LIMIT = 100 * 1024 * 1024

_U1 = [(np.arange(28)[None, :, None]
        == 2 * np.arange(12)[None, None, :] + rh
        + np.arange(5)[:, None, None]).astype(np.float32) for rh in (0, 1)]

_OH1 = [(np.arange(28)[None, :, None]
         == 2 * (np.arange(120)[None, None, :] % 12) + rw
         + np.arange(5)[:, None, None]).astype(np.float32) for rw in (0, 1)]

_U2 = (np.arange(12)[None, :, None]
       == np.arange(10)[None, None, :]
       + np.arange(3)[:, None, None]).astype(np.float32)

_OH2 = (np.arange(120)[None, :, None] % 12
        == np.arange(200)[None, None, :] % 10
        + np.arange(3)[:, None, None]).astype(np.float32)


def _build_a1f(w1):
    w1k = w1.reshape(10, 5, 5).transpose(1, 2, 0)
    e1 = jnp.repeat(w1k, 12, axis=2)
    mats = []
    for rh in (0, 1):
        u = _U1[rh].reshape(5, 336)
        for rw in (0, 1):
            a1 = jnp.einsum("hwc,wic->hic", e1, _OH1[rw])
            m = jnp.dot(u.T, a1.reshape(5, 3360))
            m = m.reshape(28, 12, 28, 120).transpose(0, 2, 1, 3)
            mats.append(m.reshape(784, 1440))
    return jnp.stack(mats, axis=0).astype(jnp.bfloat16)


def _build_a2f(w2):
    w2k = w2.reshape(20, 10, 3, 3).transpose(2, 3, 1, 0)
    e2 = jnp.repeat(jnp.repeat(w2k, 12, axis=2), 10, axis=3)
    a2 = jnp.einsum("hwrc,wrc->hrc", e2, _OH2)
    m = jnp.dot(_U2.reshape(3, 120).T, a2.reshape(3, 24000))
    m = m.reshape(12, 10, 24000).transpose(1, 0, 2)
    m = m.reshape(10, 1440, 200).transpose(1, 0, 2)
    return m.reshape(1440, 2000).astype(jnp.bfloat16)


def _lenet_kernel(x_ref, a1f_ref, b1c_ref, a2f_ref, b2c_ref, w1p_ref, bf1_ref,
                  wf2_ref, bf2_ref, o_ref):
    x = x_ref[...].astype(jnp.bfloat16)
    t = None
    for rs in range(4):
        d = jnp.dot(x, a1f_ref[rs], preferred_element_type=jnp.float32)
        t = d if t is None else jnp.maximum(t, d)
    t = jnp.maximum(t + b1c_ref[...], 0.0)
    t = t.astype(jnp.bfloat16)
    u = jnp.dot(t, a2f_ref[...], preferred_element_type=jnp.float32)
    u = jnp.maximum(u + b2c_ref[...], 0.0)
    h = jnp.dot(u.astype(jnp.bfloat16), w1p_ref[...],
                preferred_element_type=jnp.float32)
    h = jnp.maximum(h + bf1_ref[...], 0.0)
    logits = jnp.dot(h.astype(jnp.bfloat16), wf2_ref[...],
                     preferred_element_type=jnp.float32)
    logits = logits + bf2_ref[...]
    m = jnp.max(logits, axis=-1, keepdims=True)
    s = logits - m
    lse = jnp.log(jnp.sum(jnp.exp(s), axis=-1, keepdims=True))
    o_ref[...] = (s - lse).astype(o_ref.dtype)


def kernel(w1, b1, w2, b2, wf1t, bf1, wf2t, bf2, x):
    batch = x.shape[0]
    tb = 512 if batch % 512 == 0 else (128 if batch % 128 == 0 else batch)
    xf = x.reshape(batch, 28 * 28)
    a1f = _build_a1f(w1)
    a2f = _build_a2f(w2)
    b1c = jnp.tile(jnp.repeat(b1.reshape(10), 12), 12).reshape(1, 1440)
    b2c = jnp.tile(jnp.repeat(b2.reshape(20), 10), 10).reshape(1, 2000)
    w1p = (wf1t.reshape(20, 10, 10, 500).transpose(1, 0, 2, 3)
           .reshape(2000, 500).astype(jnp.bfloat16))
    wf2b = wf2t.astype(jnp.bfloat16)
    cost = pl.CostEstimate(
        flops=2 * batch * (4 * 784 * 1440 + 10 * (1440 * 200 + 200 * 500)
                           + 500 * 10),
        transcendentals=batch * 11,
        bytes_accessed=4 * (xf.size + batch * 10 + a1f.size + a2f.size
                            + w1p.size + wf2t.size),
    )
    return pl.pallas_call(
        _lenet_kernel,
        out_shape=jax.ShapeDtypeStruct((batch, 10), x.dtype),
        grid=(batch // tb,),
        in_specs=[
            pl.BlockSpec((tb, 784), lambda i: (i, 0)),
            pl.BlockSpec((4, 784, 1440), lambda i: (0, 0, 0)),
            pl.BlockSpec((1, 1440), lambda i: (0, 0)),
            pl.BlockSpec((1440, 2000), lambda i: (0, 0)),
            pl.BlockSpec((1, 2000), lambda i: (0, 0)),
            pl.BlockSpec((2000, 500), lambda i: (0, 0)),
            pl.BlockSpec((1, 500), lambda i: (0, 0)),
            pl.BlockSpec((500, 10), lambda i: (0, 0)),
            pl.BlockSpec((1, 10), lambda i: (0, 0)),
        ],
        out_specs=pl.BlockSpec((tb, 10), lambda i: (i, 0)),
        compiler_params=pltpu.CompilerParams(
            dimension_semantics=("parallel",),
            vmem_limit_bytes=_VMEM_LIMIT,
        ),
        cost_estimate=cost,
    )(xf, a1f, b1c, a2f, b2c, w1p, bf1, wf2b, bf2)


# speedup vs baseline: 1.2490x; 1.0787x over previous
import jax
import jax.numpy as jnp
import numpy as np
from jax.experimental import pallas as pl
from jax.experimental.pallas import tpu as pltpu

_VMEM_LIMIT = 100 * 1024 * 1024

_U1 = [(np.arange(28)[None, :, None]
        == 2 * np.arange(12)[None, None, :] + rh
        + np.arange(5)[:, None, None]).astype(np.float32) for rh in (0, 1)]

_OH1 = [(np.arange(28)[None, :, None]
         == 2 * (np.arange(120)[None, None, :] % 12) + rw
         + np.arange(5)[:, None, None]).astype(np.float32) for rw in (0, 1)]

_U2 = (np.arange(12)[None, :, None]
       == np.arange(10)[None, None, :]
       + np.arange(3)[:, None, None]).astype(np.float32)

_OH2 = (np.arange(120)[None, :, None] % 12
        == np.arange(200)[None, None, :] % 10
        + np.arange(3)[:, None, None]).astype(np.float32)


def _build_a1f(w1):
    w1k = w1.reshape(10, 5, 5).transpose(1, 2, 0)
    e1 = jnp.repeat(w1k, 12, axis=2)
    a1s = [jnp.einsum("hwc,wic->hic", e1, oh).reshape(5, 3360)
           for oh in _OH1]
    mats = []
    for rh in (0, 1):
        u = _U1[rh].reshape(5, 336)
        for rw in (0, 1):
            m = jnp.dot(u.T, a1s[rw])
            m = m.reshape(28, 12, 28, 120).transpose(0, 2, 1, 3)
            mats.append(m.reshape(784, 1440))
    return jnp.stack(mats, axis=0).astype(jnp.bfloat16)


def _build_a2f(w2):
    w2k = w2.reshape(20, 10, 3, 3).transpose(2, 3, 1, 0)
    e2 = jnp.repeat(jnp.repeat(w2k, 12, axis=2), 10, axis=3)
    a2 = jnp.einsum("hwrc,wrc->hrc", e2, _OH2)
    m = jnp.dot(_U2.reshape(3, 120).T, a2.reshape(3, 24000))
    m = m.reshape(12, 10, 120, 200).transpose(0, 2, 1, 3)
    return m.reshape(1440, 2000).astype(jnp.bfloat16)


def _lenet_kernel(x_ref, a1f_ref, b1c_ref, a2f_ref, b2c_ref, w1p_ref, bf1_ref,
                  wf2_ref, bf2_ref, o_ref):
    x = x_ref[...].astype(jnp.bfloat16)
    t = None
    for rs in range(4):
        d = jnp.dot(x, a1f_ref[rs], preferred_element_type=jnp.float32)
        t = d if t is None else jnp.maximum(t, d)
    t = jnp.maximum(t + b1c_ref[...], 0.0)
    t = t.astype(jnp.bfloat16)
    u = jnp.dot(t, a2f_ref[...], preferred_element_type=jnp.float32)
    u = jnp.maximum(u + b2c_ref[...], 0.0)
    h = jnp.dot(u.astype(jnp.bfloat16), w1p_ref[...],
                preferred_element_type=jnp.float32)
    h = jnp.maximum(h + bf1_ref[...], 0.0)
    logits = jnp.dot(h.astype(jnp.bfloat16), wf2_ref[...],
                     preferred_element_type=jnp.float32)
    logits = logits + bf2_ref[...]
    m = jnp.max(logits, axis=-1, keepdims=True)
    s = logits - m
    lse = jnp.log(jnp.sum(jnp.exp(s), axis=-1, keepdims=True))
    o_ref[...] = (s - lse).astype(o_ref.dtype)


def kernel(w1, b1, w2, b2, wf1t, bf1, wf2t, bf2, x):
    batch = x.shape[0]
    tb = 704 if batch % 704 == 0 else (128 if batch % 128 == 0 else batch)
    xf = x.reshape(batch, 28 * 28)
    a1f = _build_a1f(w1)
    a2f = _build_a2f(w2)
    b1c = jnp.tile(jnp.repeat(b1.reshape(10), 12), 12).reshape(1, 1440)
    b2c = jnp.tile(jnp.repeat(b2.reshape(20), 10), 10).reshape(1, 2000)
    w1p = (wf1t.reshape(20, 10, 10, 500).transpose(1, 0, 2, 3)
           .reshape(2000, 500).astype(jnp.bfloat16))
    wf2b = wf2t.astype(jnp.bfloat16)
    cost = pl.CostEstimate(
        flops=2 * batch * (4 * 784 * 1440 + 10 * (1440 * 200 + 200 * 500)
                           + 500 * 10),
        transcendentals=batch * 11,
        bytes_accessed=4 * (xf.size + batch * 10 + a1f.size + a2f.size
                            + w1p.size + wf2t.size),
    )
    return pl.pallas_call(
        _lenet_kernel,
        out_shape=jax.ShapeDtypeStruct((batch, 10), x.dtype),
        grid=(batch // tb,),
        in_specs=[
            pl.BlockSpec((tb, 784), lambda i: (i, 0)),
            pl.BlockSpec((4, 784, 1440), lambda i: (0, 0, 0)),
            pl.BlockSpec((1, 1440), lambda i: (0, 0)),
            pl.BlockSpec((1440, 2000), lambda i: (0, 0)),
            pl.BlockSpec((1, 2000), lambda i: (0, 0)),
            pl.BlockSpec((2000, 500), lambda i: (0, 0)),
            pl.BlockSpec((1, 500), lambda i: (0, 0)),
            pl.BlockSpec((500, 10), lambda i: (0, 0)),
            pl.BlockSpec((1, 10), lambda i: (0, 0)),
        ],
        out_specs=pl.BlockSpec((tb, 10), lambda i: (i, 0)),
        compiler_params=pltpu.CompilerParams(
            dimension_semantics=("parallel",),
            vmem_limit_bytes=_VMEM_LIMIT,
        ),
        cost_estimate=cost,
    )(xf, a1f, b1c, a2f, b2c, w1p, bf1, wf2b, bf2)
